# baseline (speedup 1.0000x reference)
import jax
import jax.numpy as jnp
from jax import lax
from jax.experimental import pallas as pl
from jax.experimental.pallas import tpu as pltpu
from jax.experimental.pallas import tpu_sc as plsc

N_NODES = 10000
N_EDGES = 320000
IN_CH = 128
HID = 64
OUT_CH = 128

NC = 2
NS = 16
NW = NC * NS

CHUNK = 128
E_PAD = 327680
NCHUNKS = E_PAD // CHUNK
CPT1 = NCHUNKS // NW
CPT2 = NCHUNKS // NS
N_PAD = 10112
ROWS_PER_TILE = N_PAD // NS
D = 64
NBUF = 4


def _sc_mesh():
  return plsc.VectorSubcoreMesh(core_axis_name="c", subcore_axis_name="s")


def _newton_rsqrt(d):
  i = plsc.bitcast(d, jnp.int32)
  i = 0x5F3759DF - lax.shift_right_arithmetic(i, 1)
  y = plsc.bitcast(i, jnp.float32)
  for _ in range(3):
    y = y * (1.5 - 0.5 * d * y * y)
  return y


def _zero_table(zrows_v, table_sh, row0):
  for z0 in (0, 128, 256, 384):
    pltpu.sync_copy(zrows_v, table_sh.at[pl.ds(row0 + z0, 128)])
  pltpu.sync_copy(zrows_v.at[pl.ds(0, ROWS_PER_TILE - 512)],
                  table_sh.at[pl.ds(row0 + 512, ROWS_PER_TILE - 512)])


def _deg_body(dstc_hbm, ones_hbm, zrows_hbm, dinv_hbm,
              idx_v, ones_v, zrows_v, deg_v, table_sh, sa, sb, osem):
  c = lax.axis_index("c")
  s = lax.axis_index("s")
  row0 = s * ROWS_PER_TILE

  pltpu.sync_copy(zrows_hbm, zrows_v)
  _zero_table(zrows_v, table_sh, row0)
  pltpu.sync_copy(ones_hbm, ones_v)
  pltpu.sync_copy(dstc_hbm.at[pl.ds(s * CPT2, CPT2)], idx_v)
  plsc.subcore_barrier()

  def fire(k, sem):
    pltpu.async_copy(ones_v, table_sh.at[idx_v.at[k]], sem, add=True)

  def drain(k, sem):
    pltpu.make_async_copy(ones_v, table_sh.at[idx_v.at[k]], sem).wait()

  fire(0, sa)

  @pl.loop(0, CPT2, step=2)
  def _chunks(k0):
    fire(k0 + 1, sb)
    drain(k0, sa)

    @pl.when(k0 + 2 < CPT2)
    def _pre():
      fire(k0 + 2, sa)

    drain(k0 + 1, sb)

  plsc.subcore_barrier()

  pltpu.sync_copy(table_sh.at[pl.ds(row0, ROWS_PER_TILE)], deg_v)

  @pl.loop(0, ROWS_PER_TILE)
  def _nr(r):
    deg_v[r] = _newton_rsqrt(deg_v[r] + 1.0)

  @pl.when(c == 0)
  def _writeout():
    pltpu.async_copy(deg_v, dinv_hbm.at[pl.ds(row0, ROWS_PER_TILE)],
                     osem).wait()


def _deg_pass(dstc, ones, zrows):
  kfn = pl.kernel(
      _deg_body,
      out_type=jax.ShapeDtypeStruct((N_PAD, 16), jnp.float32),
      mesh=_sc_mesh(),
      scratch_types=[
          pltpu.VMEM((CPT2, CHUNK), jnp.int32),
          pltpu.VMEM((CHUNK, 16), jnp.float32),
          pltpu.VMEM((128, 16), jnp.float32),
          pltpu.VMEM((ROWS_PER_TILE, 16), jnp.float32),
          pltpu.VMEM_SHARED((N_PAD, 16), jnp.float32),
          pltpu.SemaphoreType.DMA,
          pltpu.SemaphoreType.DMA,
          pltpu.SemaphoreType.DMA,
      ],
      compiler_params=pltpu.CompilerParams(use_tc_tiling_on_sc=False, needs_layout_passes=False),
  )
  return kfn(dstc, ones, zrows)


def _edge_loop(ys_ref, src_v, dst_v, rows_v, table_sh, gsems, ssems, cpt):
  def gather(k, b):
    pltpu.async_copy(ys_ref.at[src_v.at[k]], rows_v.at[b], gsems[b])

  def wait_gather(k, b):
    pltpu.make_async_copy(ys_ref.at[src_v.at[k]], rows_v.at[b],
                          gsems[b]).wait()

  def scatter(k, b):
    pltpu.async_copy(rows_v.at[b], table_sh.at[dst_v.at[k]], ssems[b],
                     add=True)

  def wait_scatter(k, b):
    pltpu.make_async_copy(rows_v.at[b], table_sh.at[dst_v.at[k]],
                          ssems[b]).wait()

  gather(0, 0)
  gather(1, 1)

  @pl.loop(0, cpt, step=NBUF)
  def _chunks(k0):
    for b in range(NBUF):
      k = k0 + b
      bp2 = (b + 2) % NBUF

      @pl.when(k >= 2)
      def _wait_prev_scatter():
        wait_scatter(k - 2, bp2)

      @pl.when(k + 2 < cpt)
      def _prefetch():
        gather(k + 2, bp2)

      wait_gather(k, b)
      scatter(k, b)

  wait_scatter(cpt - 2, (cpt - 2) % NBUF)
  wait_scatter(cpt - 1, (cpt - 1) % NBUF)


def _row_chunks(row0):
  return [(row0, 128), (row0 + 128, 128), (row0 + 256, 128),
          (row0 + 384, 128), (row0 + 512, ROWS_PER_TILE - 512)]


def _edge1_body(xw_hbm, dinv_hbm, srcc_hbm, dstc_hbm, zrows_hbm,
                p_out, ys1_out,
                src_v, dst_v, rows_v, zrows_v, dinv_v, table_sh,
                g0, g1, g2, g3, s0, s1, s2, s3, osem):
  c = lax.axis_index("c")
  s = lax.axis_index("s")
  wid = s * NC + c
  row0 = s * ROWS_PER_TILE

  pltpu.sync_copy(zrows_hbm, zrows_v)
  _zero_table(zrows_v, table_sh, row0)
  pltpu.sync_copy(srcc_hbm.at[pl.ds(wid * CPT1, CPT1)], src_v)
  pltpu.sync_copy(dstc_hbm.at[pl.ds(wid * CPT1, CPT1)], dst_v)

  for r0, nr in _row_chunks(row0):
    pltpu.sync_copy(xw_hbm.at[pl.ds(r0, nr)], rows_v.at[0, pl.ds(0, nr)])
    pltpu.sync_copy(dinv_hbm.at[pl.ds(r0, nr)], dinv_v.at[pl.ds(0, nr)])

    @pl.loop(0, nr)
    def _scale(r):
      dv = dinv_v[r]
      for j in range(D // 16):
        rows_v[1, r, pl.ds(j * 16, 16)] = (
            rows_v[0, r, pl.ds(j * 16, 16)] * dv)

    pltpu.sync_copy(rows_v.at[1, pl.ds(0, nr)], ys1_out.at[pl.ds(r0, nr)])

  plsc.subcore_barrier()

  _edge_loop(ys1_out, src_v, dst_v, rows_v, table_sh,
             (g0, g1, g2, g3), (s0, s1, s2, s3), CPT1)
  plsc.subcore_barrier()

  pltpu.async_copy(
      table_sh.at[pl.ds(row0, ROWS_PER_TILE)],
      p_out.at[c, pl.ds(row0, ROWS_PER_TILE)],
      osem,
  ).wait()


def _edge_pass1(xw1, dinv_t, srcc, dstc, zrows):
  kfn = pl.kernel(
      _edge1_body,
      out_type=(
          jax.ShapeDtypeStruct((NC, N_PAD, D), jnp.float32),
          jax.ShapeDtypeStruct((N_PAD, D), jnp.float32),
      ),
      mesh=_sc_mesh(),
      scratch_types=[
          pltpu.VMEM((CPT1, CHUNK), jnp.int32),
          pltpu.VMEM((CPT1, CHUNK), jnp.int32),
          pltpu.VMEM((NBUF, CHUNK, D), jnp.float32),
          pltpu.VMEM((128, D), jnp.float32),
          pltpu.VMEM((128, 16), jnp.float32),
          pltpu.VMEM_SHARED((N_PAD, D), jnp.float32),
      ] + [pltpu.SemaphoreType.DMA] * 9,
      compiler_params=pltpu.CompilerParams(use_tc_tiling_on_sc=False, needs_layout_passes=False),
  )
  return kfn(xw1, dinv_t, srcc, dstc, zrows)


def _edge2_body(ys3_hbm, dinv_hbm, srcc_hbm, dstc_hbm, zrows_hbm, b2_hbm,
                out_hbm,
                src_v, dst_v, rows_v, zrows_v, dinv_v, b_v, table_sh,
                g0, g1, g2, g3, s0, s1, s2, s3, osem):
  c = lax.axis_index("c")
  s = lax.axis_index("s")
  row0 = s * ROWS_PER_TILE

  pltpu.sync_copy(zrows_hbm, zrows_v)
  _zero_table(zrows_v, table_sh, row0)
  pltpu.sync_copy(srcc_hbm.at[pl.ds(s * CPT2, CPT2)], src_v)
  pltpu.sync_copy(dstc_hbm.at[pl.ds(s * CPT2, CPT2)], dst_v)
  pltpu.sync_copy(b2_hbm.at[c], b_v)
  plsc.subcore_barrier()

  _edge_loop(ys3_hbm.at[c], src_v, dst_v, rows_v, table_sh,
             (g0, g1, g2, g3), (s0, s1, s2, s3), CPT2)
  plsc.subcore_barrier()

  for r0, nr in _row_chunks(row0):
    pltpu.sync_copy(table_sh.at[pl.ds(r0, nr)], rows_v.at[0, pl.ds(0, nr)])
    pltpu.sync_copy(ys3_hbm.at[c, pl.ds(r0, nr)], rows_v.at[1, pl.ds(0, nr)])
    pltpu.sync_copy(dinv_hbm.at[pl.ds(r0, nr)], dinv_v.at[pl.ds(0, nr)])

    @pl.loop(0, nr)
    def _finish(r):
      dv = dinv_v[r]
      for j in range(D // 16):
        sl = pl.ds(j * 16, 16)
        rows_v[2, r, sl] = (
            dv * (rows_v[0, r, sl] + rows_v[1, r, sl]) + b_v[j])

    pltpu.sync_copy(rows_v.at[2, pl.ds(0, nr)], out_hbm.at[c, pl.ds(r0, nr)])


def _edge_pass2(ys3, dinv_t, srcc, dstc, zrows, b2r):
  kfn = pl.kernel(
      _edge2_body,
      out_type=jax.ShapeDtypeStruct((NC, N_PAD, D), jnp.float32),
      mesh=_sc_mesh(),
      scratch_types=[
          pltpu.VMEM((CPT2, CHUNK), jnp.int32),
          pltpu.VMEM((CPT2, CHUNK), jnp.int32),
          pltpu.VMEM((NBUF, CHUNK, D), jnp.float32),
          pltpu.VMEM((128, D), jnp.float32),
          pltpu.VMEM((128, 16), jnp.float32),
          pltpu.VMEM((D // 16, 16), jnp.float32),
          pltpu.VMEM_SHARED((N_PAD, D), jnp.float32),
      ] + [pltpu.SemaphoreType.DMA] * 9,
      compiler_params=pltpu.CompilerParams(use_tc_tiling_on_sc=False, needs_layout_passes=False),
  )
  return kfn(ys3, dinv_t, srcc, dstc, zrows, b2r)


def _tc_matmul_body(x_ref, w_ref, xw_ref):
  xw_ref[...] = jnp.dot(x_ref[...], w_ref[...],
                        preferred_element_type=jnp.float32)


def _tc_matmul(x_pad, w):
  return pl.pallas_call(
      _tc_matmul_body,
      out_shape=jax.ShapeDtypeStruct((N_PAD, w.shape[1]), jnp.float32),
  )(x_pad, w)


def _tc_layer1_finish_body(p_ref, ys_ref, dinv_ref, b_ref, w2_ref, ys3_ref):
  agg = p_ref[0] + p_ref[1] + ys_ref[...]
  h = jnp.maximum(agg * dinv_ref[...] + b_ref[...], 0.0)
  hw = jnp.dot(h, w2_ref[...], preferred_element_type=jnp.float32)
  ys2 = hw * dinv_ref[...]
  ys3_ref[0] = ys2[:, :D]
  ys3_ref[1] = ys2[:, D:]


def _tc_layer1_finish(partials, ys1, dinv_col, b1, w2):
  return pl.pallas_call(
      _tc_layer1_finish_body,
      out_shape=jax.ShapeDtypeStruct((NC, N_PAD, D), jnp.float32),
  )(partials, ys1, dinv_col, b1.reshape(1, HID), w2)


def _gcn_block(x, edge_index, W1, b1, W2, b2):
  src = edge_index[0].astype(jnp.int32)
  dst = edge_index[1].astype(jnp.int32)
  npad_e = E_PAD - N_EDGES
  pad_idx = N_NODES + (jnp.arange(npad_e, dtype=jnp.int32) % (N_PAD - N_NODES))
  srcc = jnp.concatenate([src, pad_idx]).reshape(NCHUNKS, CHUNK)
  dstc = jnp.concatenate([dst, pad_idx]).reshape(NCHUNKS, CHUNK)

  x_pad = jnp.zeros((N_PAD, IN_CH), x.dtype).at[:N_NODES].set(x)
  b2r = b2.reshape(NC, D // 16, 16)

  ones = jnp.ones((CHUNK, 16), jnp.float32)
  zrows16 = jnp.zeros((128, 16), jnp.float32)
  zrows_d = jnp.zeros((128, D), jnp.float32)

  dinv_t = _deg_pass(dstc, ones, zrows16)
  xw1 = _tc_matmul(x_pad, W1)

  p1, ys1 = _edge_pass1(xw1, dinv_t, srcc, dstc, zrows_d)
  ys3 = _tc_layer1_finish(p1, ys1, dinv_t[:, 0:1], b1, W2)
  outh = _edge_pass2(ys3, dinv_t, srcc, dstc, zrows_d, b2r)

  out = jnp.concatenate([outh[0], outh[1]], axis=1)
  return out[:N_NODES]


def kernel(x, edge_index, W1, b1, W2, b2):
  return _gcn_block(x, edge_index, W1, b1, W2, b2)


# speedup vs baseline: 1.0456x; 1.0456x over previous
import jax
import jax.numpy as jnp
from jax import lax
from jax.experimental import pallas as pl
from jax.experimental.pallas import tpu as pltpu
from jax.experimental.pallas import tpu_sc as plsc

N_NODES = 10000
N_EDGES = 320000
IN_CH = 128
HID = 64
OUT_CH = 128

NC = 2
NS = 16
NW = NC * NS

CHUNK = 128
EPT = 10240
E_PAD = EPT * NW
NCHUNKS = E_PAD // CHUNK
CPT1 = NCHUNKS // NW
CPT2 = NCHUNKS // NS
N_PAD = 10112
ROWS_PER_TILE = N_PAD // NS
D = 64
NBUF = 4


def _sc_mesh():
  return plsc.VectorSubcoreMesh(core_axis_name="c", subcore_axis_name="s")


def _zero_table_and_preload(zrows_hbm, zrows_v, table_sh, s, psem, copies):
  del psem
  pltpu.sync_copy(zrows_hbm, zrows_v)
  row0 = s * ROWS_PER_TILE
  for z0 in (0, 128, 256, 384):
    pltpu.sync_copy(zrows_v, table_sh.at[pl.ds(row0 + z0, 128)])
  pltpu.sync_copy(zrows_v.at[pl.ds(0, ROWS_PER_TILE - 512)],
                  table_sh.at[pl.ds(row0 + 512, ROWS_PER_TILE - 512)])
  for src, dst in copies:
    pltpu.sync_copy(src, dst)
  return row0


def _deg_body(dstc_hbm, ones_hbm, zrows_hbm, out_hbm,
              idx_v, ones_v, zrows_v, table_sh, psem, ssem, osem):
  c = lax.axis_index("c")
  s = lax.axis_index("s")
  wid = s * NC + c

  row0 = _zero_table_and_preload(
      zrows_hbm, zrows_v, table_sh, s, psem,
      [(ones_hbm, ones_v),
       (dstc_hbm.at[pl.ds(wid * CPT1, CPT1)], idx_v)])
  plsc.subcore_barrier()

  @pl.loop(0, CPT1)
  def _fire(k):
    pltpu.async_copy(ones_v, table_sh.at[idx_v.at[k]], ssem, add=True).wait()

  plsc.subcore_barrier()

  pltpu.async_copy(
      table_sh.at[pl.ds(row0, ROWS_PER_TILE)],
      out_hbm.at[c, pl.ds(row0, ROWS_PER_TILE)],
      osem,
  ).wait()


def _deg_pass(dstc, ones, zrows):
  kfn = pl.kernel(
      _deg_body,
      out_type=jax.ShapeDtypeStruct((NC, N_PAD, 8), jnp.float32),
      mesh=_sc_mesh(),
      scratch_types=[
          pltpu.VMEM((CPT1, CHUNK), jnp.int32),
          pltpu.VMEM((CHUNK, 8), jnp.float32),
          pltpu.VMEM((128, 8), jnp.float32),
          pltpu.VMEM_SHARED((N_PAD, 8), jnp.float32),
          pltpu.SemaphoreType.DMA,
          pltpu.SemaphoreType.DMA,
          pltpu.SemaphoreType.DMA,
      ],
      compiler_params=pltpu.CompilerParams(use_tc_tiling_on_sc=False),
  )
  return kfn(dstc, ones, zrows)


def _edge_loop(ys_ref, src_v, dst_v, rows_v, table_sh, gsems, ssems, cpt):
  def gather(k, b):
    pltpu.async_copy(ys_ref.at[src_v.at[k]], rows_v.at[b], gsems[b])

  def wait_gather(k, b):
    pltpu.make_async_copy(ys_ref.at[src_v.at[k]], rows_v.at[b],
                          gsems[b]).wait()

  def scatter(k, b):
    pltpu.async_copy(rows_v.at[b], table_sh.at[dst_v.at[k]], ssems[b],
                     add=True)

  def wait_scatter(k, b):
    pltpu.make_async_copy(rows_v.at[b], table_sh.at[dst_v.at[k]],
                          ssems[b]).wait()

  gather(0, 0)
  gather(1, 1)

  @pl.loop(0, cpt, step=NBUF)
  def _chunks(k0):
    for b in range(NBUF):
      k = k0 + b
      bp2 = (b + 2) % NBUF

      @pl.when(k >= 2)
      def _wait_prev_scatter():
        wait_scatter(k - 2, bp2)

      @pl.when(k + 2 < cpt)
      def _prefetch():
        gather(k + 2, bp2)

      wait_gather(k, b)
      scatter(k, b)

  wait_scatter(cpt - 2, (cpt - 2) % NBUF)
  wait_scatter(cpt - 1, (cpt - 1) % NBUF)


def _edge1_body(ys_hbm, srcc_hbm, dstc_hbm, zrows_hbm, out_hbm,
                src_v, dst_v, rows_v, zrows_v, table_sh,
                g0, g1, g2, g3, s0, s1, s2, s3, psem, osem):
  c = lax.axis_index("c")
  s = lax.axis_index("s")
  wid = s * NC + c

  row0 = _zero_table_and_preload(
      zrows_hbm, zrows_v, table_sh, s, psem,
      [(srcc_hbm.at[pl.ds(wid * CPT1, CPT1)], src_v),
       (dstc_hbm.at[pl.ds(wid * CPT1, CPT1)], dst_v)])
  plsc.subcore_barrier()

  _edge_loop(ys_hbm, src_v, dst_v, rows_v, table_sh,
             (g0, g1, g2, g3), (s0, s1, s2, s3), CPT1)
  plsc.subcore_barrier()

  pltpu.async_copy(
      table_sh.at[pl.ds(row0, ROWS_PER_TILE)],
      out_hbm.at[c, pl.ds(row0, ROWS_PER_TILE)],
      osem,
  ).wait()


def _edge_pass1(ys_pad, srcc, dstc, zrows):
  kfn = pl.kernel(
      _edge1_body,
      out_type=jax.ShapeDtypeStruct((NC, N_PAD, D), jnp.float32),
      mesh=_sc_mesh(),
      scratch_types=[
          pltpu.VMEM((CPT1, CHUNK), jnp.int32),
          pltpu.VMEM((CPT1, CHUNK), jnp.int32),
          pltpu.VMEM((NBUF, CHUNK, D), jnp.float32),
          pltpu.VMEM((128, D), jnp.float32),
          pltpu.VMEM_SHARED((N_PAD, D), jnp.float32),
      ] + [pltpu.SemaphoreType.DMA] * 10,
      compiler_params=pltpu.CompilerParams(use_tc_tiling_on_sc=False),
  )
  return kfn(ys_pad, srcc, dstc, zrows)


def _edge2_body(ys3_hbm, dinv16_hbm, srcc_hbm, dstc_hbm, zrows_hbm, b2_hbm,
                out_hbm,
                src_v, dst_v, rows_v, zrows_v, dinv_v, b_v, table_sh,
                g0, g1, g2, g3, s0, s1, s2, s3, psem, osem):
  c = lax.axis_index("c")
  s = lax.axis_index("s")

  row0 = _zero_table_and_preload(
      zrows_hbm, zrows_v, table_sh, s, psem,
      [(srcc_hbm.at[pl.ds(s * CPT2, CPT2)], src_v),
       (dstc_hbm.at[pl.ds(s * CPT2, CPT2)], dst_v),
       (b2_hbm.at[c], b_v)])
  plsc.subcore_barrier()

  _edge_loop(ys3_hbm.at[c], src_v, dst_v, rows_v, table_sh,
             (g0, g1, g2, g3), (s0, s1, s2, s3), CPT2)
  plsc.subcore_barrier()

  for r0, nr in ((row0, 128), (row0 + 128, 128), (row0 + 256, 128),
                 (row0 + 384, 128), (row0 + 512, ROWS_PER_TILE - 512)):
    pltpu.sync_copy(table_sh.at[pl.ds(r0, nr)], rows_v.at[0, pl.ds(0, nr)])
    pltpu.sync_copy(ys3_hbm.at[c, pl.ds(r0, nr)], rows_v.at[1, pl.ds(0, nr)])
    pltpu.sync_copy(dinv16_hbm.at[pl.ds(r0, nr)], dinv_v.at[pl.ds(0, nr)])

    @pl.loop(0, nr, unroll=8)
    def _finish(r):
      dv = dinv_v[r]
      for j in range(D // 16):
        sl = pl.ds(j * 16, 16)
        rows_v[2, r, sl] = (
            dv * (rows_v[0, r, sl] + rows_v[1, r, sl]) + b_v[j])

    pltpu.sync_copy(rows_v.at[2, pl.ds(0, nr)], out_hbm.at[c, pl.ds(r0, nr)])


def _edge_pass2(ys3, dinv16, srcc, dstc, zrows, b2r):
  kfn = pl.kernel(
      _edge2_body,
      out_type=jax.ShapeDtypeStruct((NC, N_PAD, D), jnp.float32),
      mesh=_sc_mesh(),
      scratch_types=[
          pltpu.VMEM((CPT2, CHUNK), jnp.int32),
          pltpu.VMEM((CPT2, CHUNK), jnp.int32),
          pltpu.VMEM((NBUF, CHUNK, D), jnp.float32),
          pltpu.VMEM((128, D), jnp.float32),
          pltpu.VMEM((128, 16), jnp.float32),
          pltpu.VMEM((D // 16, 16), jnp.float32),
          pltpu.VMEM_SHARED((N_PAD, D), jnp.float32),
      ] + [pltpu.SemaphoreType.DMA] * 10,
      compiler_params=pltpu.CompilerParams(use_tc_tiling_on_sc=False),
  )
  return kfn(ys3, dinv16, srcc, dstc, zrows, b2r)


def _tc_scale_matmul_body(degp_ref, x_ref, w_ref, ys_ref, dinv_ref,
                          dinv16_ref):
  deg = degp_ref[0] + degp_ref[1] + 1.0
  dinv = lax.rsqrt(deg)
  dinv_ref[...] = dinv
  dinv16_ref[...] = jnp.broadcast_to(dinv, (N_PAD, 16))
  xw = jnp.dot(x_ref[...], w_ref[...], preferred_element_type=jnp.float32)
  ys_ref[...] = xw * dinv


def _tc_scale_matmul(degp, x_pad, w):
  d_out = w.shape[1]
  deg_cols = degp[:, :, 0:1]
  return pl.pallas_call(
      _tc_scale_matmul_body,
      out_shape=(
          jax.ShapeDtypeStruct((N_PAD, d_out), jnp.float32),
          jax.ShapeDtypeStruct((N_PAD, 1), jnp.float32),
          jax.ShapeDtypeStruct((N_PAD, 16), jnp.float32),
      ),
  )(deg_cols, x_pad, w)


def _tc_layer1_finish_body(p_ref, ys_ref, dinv_ref, b_ref, w2_ref, ys3_ref):
  agg = p_ref[0] + p_ref[1] + ys_ref[...]
  h = jnp.maximum(agg * dinv_ref[...] + b_ref[...], 0.0)
  hw = jnp.dot(h, w2_ref[...], preferred_element_type=jnp.float32)
  ys2 = hw * dinv_ref[...]
  ys3_ref[0] = ys2[:, :D]
  ys3_ref[1] = ys2[:, D:]


def _tc_layer1_finish(partials, ys1, dinv, b1, w2):
  return pl.pallas_call(
      _tc_layer1_finish_body,
      out_shape=jax.ShapeDtypeStruct((NC, N_PAD, D), jnp.float32),
  )(partials, ys1, dinv, b1.reshape(1, HID), w2)


def _gcn_block(x, edge_index, W1, b1, W2, b2):
  src = edge_index[0].astype(jnp.int32)
  dst = edge_index[1].astype(jnp.int32)
  npad_e = E_PAD - N_EDGES
  pad_idx = N_NODES + (jnp.arange(npad_e, dtype=jnp.int32) % (N_PAD - N_NODES))
  srcc = jnp.concatenate([src, pad_idx]).reshape(NCHUNKS, CHUNK)
  dstc = jnp.concatenate([dst, pad_idx]).reshape(NCHUNKS, CHUNK)

  x_pad = jnp.zeros((N_PAD, IN_CH), x.dtype).at[:N_NODES].set(x)

  ones = jnp.ones((CHUNK, 8), jnp.float32)
  zrows8 = jnp.zeros((128, 8), jnp.float32)
  zrows_d = jnp.zeros((128, D), jnp.float32)

  b2r = b2.reshape(NC, D // 16, 16)
  degp = _deg_pass(dstc, ones, zrows8)

  ys1, dinv, dinv16 = _tc_scale_matmul(degp, x_pad, W1)
  p1 = _edge_pass1(ys1, srcc, dstc, zrows_d)

  ys3 = _tc_layer1_finish(p1, ys1, dinv, b1, W2)
  outh = _edge_pass2(ys3, dinv16, srcc, dstc, zrows_d, b2r)

  out = jnp.concatenate([outh[0], outh[1]], axis=1)
  return out[:N_NODES]


def kernel(x, edge_index, W1, b1, W2, b2):
  return _gcn_block(x, edge_index, W1, b1, W2, b2)


# speedup vs baseline: 1.1264x; 1.0772x over previous
import jax
import jax.numpy as jnp
from jax import lax
from jax.experimental import pallas as pl
from jax.experimental.pallas import tpu as pltpu
from jax.experimental.pallas import tpu_sc as plsc

N_NODES = 10000
N_EDGES = 320000
IN_CH = 128
HID = 64
OUT_CH = 128

NC = 2
NS = 16
NW = NC * NS

CHUNK = 128
EPT = 10240
E_PAD = EPT * NW
NCHUNKS = E_PAD // CHUNK
CPT1 = NCHUNKS // NW
CPT2 = NCHUNKS // NS
N_PAD = 10112
ROWS_PER_TILE = N_PAD // NS
D = 64
NBUF = 4


def _sc_mesh():
  return plsc.VectorSubcoreMesh(core_axis_name="c", subcore_axis_name="s")


def _zero_table_and_preload(zrows_hbm, zrows_v, table_sh, s, psem, copies):
  del psem
  pltpu.sync_copy(zrows_hbm, zrows_v)
  row0 = s * ROWS_PER_TILE
  for z0 in (0, 128, 256, 384):
    pltpu.sync_copy(zrows_v, table_sh.at[pl.ds(row0 + z0, 128)])
  pltpu.sync_copy(zrows_v.at[pl.ds(0, ROWS_PER_TILE - 512)],
                  table_sh.at[pl.ds(row0 + 512, ROWS_PER_TILE - 512)])
  for src, dst in copies:
    pltpu.sync_copy(src, dst)
  return row0


def _deg_body(dstc_hbm, ones_hbm, zrows_hbm, out_hbm,
              idx_v, ones_v, zrows_v, table_sh, psem, ssem, osem):
  c = lax.axis_index("c")
  s = lax.axis_index("s")
  wid = s * NC + c

  row0 = _zero_table_and_preload(
      zrows_hbm, zrows_v, table_sh, s, psem,
      [(ones_hbm, ones_v),
       (dstc_hbm.at[pl.ds(wid * CPT1, CPT1)], idx_v)])
  plsc.subcore_barrier()

  @pl.loop(0, CPT1)
  def _fire(k):
    pltpu.async_copy(ones_v, table_sh.at[idx_v.at[k]], ssem, add=True).wait()

  plsc.subcore_barrier()

  pltpu.async_copy(
      table_sh.at[pl.ds(row0, ROWS_PER_TILE)],
      out_hbm.at[c, pl.ds(row0, ROWS_PER_TILE)],
      osem,
  ).wait()


def _deg_pass(dstc, ones, zrows):
  kfn = pl.kernel(
      _deg_body,
      out_type=jax.ShapeDtypeStruct((NC, N_PAD, 8), jnp.float32),
      mesh=_sc_mesh(),
      scratch_types=[
          pltpu.VMEM((CPT1, CHUNK), jnp.int32),
          pltpu.VMEM((CHUNK, 8), jnp.float32),
          pltpu.VMEM((128, 8), jnp.float32),
          pltpu.VMEM_SHARED((N_PAD, 8), jnp.float32),
          pltpu.SemaphoreType.DMA,
          pltpu.SemaphoreType.DMA,
          pltpu.SemaphoreType.DMA,
      ],
      compiler_params=pltpu.CompilerParams(use_tc_tiling_on_sc=False),
  )
  return kfn(dstc, ones, zrows)


def _edge_loop(ys_ref, src_v, dst_v, rows_v, table_sh, gsems, ssems, cpt):
  def gather(k, b):
    pltpu.async_copy(ys_ref.at[src_v.at[k]], rows_v.at[b], gsems[b])

  def wait_gather(k, b):
    pltpu.make_async_copy(ys_ref.at[src_v.at[k]], rows_v.at[b],
                          gsems[b]).wait()

  def scatter(k, b):
    pltpu.async_copy(rows_v.at[b], table_sh.at[dst_v.at[k]], ssems[b],
                     add=True)

  def wait_scatter(k, b):
    pltpu.make_async_copy(rows_v.at[b], table_sh.at[dst_v.at[k]],
                          ssems[b]).wait()

  gather(0, 0)
  gather(1, 1)

  @pl.loop(0, cpt, step=NBUF)
  def _chunks(k0):
    for b in range(NBUF):
      k = k0 + b
      bp2 = (b + 2) % NBUF

      @pl.when(k >= 2)
      def _wait_prev_scatter():
        wait_scatter(k - 2, bp2)

      @pl.when(k + 2 < cpt)
      def _prefetch():
        gather(k + 2, bp2)

      wait_gather(k, b)
      scatter(k, b)

  wait_scatter(cpt - 2, (cpt - 2) % NBUF)
  wait_scatter(cpt - 1, (cpt - 1) % NBUF)


def _edge1_body(ys_hbm, srcc_hbm, dstc_hbm, zrows_hbm, out_hbm,
                src_v, dst_v, rows_v, zrows_v, table_sh,
                g0, g1, g2, g3, s0, s1, s2, s3, psem, osem):
  c = lax.axis_index("c")
  s = lax.axis_index("s")
  wid = s * NC + c

  row0 = _zero_table_and_preload(
      zrows_hbm, zrows_v, table_sh, s, psem,
      [(srcc_hbm.at[pl.ds(wid * CPT1, CPT1)], src_v),
       (dstc_hbm.at[pl.ds(wid * CPT1, CPT1)], dst_v)])
  plsc.subcore_barrier()

  _edge_loop(ys_hbm, src_v, dst_v, rows_v, table_sh,
             (g0, g1, g2, g3), (s0, s1, s2, s3), CPT1)
  plsc.subcore_barrier()

  pltpu.async_copy(
      table_sh.at[pl.ds(row0, ROWS_PER_TILE)],
      out_hbm.at[c, pl.ds(row0, ROWS_PER_TILE)],
      osem,
  ).wait()


def _edge_pass1(ys_pad, srcc, dstc, zrows):
  kfn = pl.kernel(
      _edge1_body,
      out_type=jax.ShapeDtypeStruct((NC, N_PAD, D), jnp.float32),
      mesh=_sc_mesh(),
      scratch_types=[
          pltpu.VMEM((CPT1, CHUNK), jnp.int32),
          pltpu.VMEM((CPT1, CHUNK), jnp.int32),
          pltpu.VMEM((NBUF, CHUNK, D), jnp.float32),
          pltpu.VMEM((128, D), jnp.float32),
          pltpu.VMEM_SHARED((N_PAD, D), jnp.float32),
      ] + [pltpu.SemaphoreType.DMA] * 10,
      compiler_params=pltpu.CompilerParams(use_tc_tiling_on_sc=False),
  )
  return kfn(ys_pad, srcc, dstc, zrows)


def _edge2_body(ys3_hbm, srcc_hbm, dstc_hbm, zrows_hbm, out_hbm,
                src_v, dst_v, rows_v, zrows_v, table_sh,
                g0, g1, g2, g3, s0, s1, s2, s3, psem, osem):
  c = lax.axis_index("c")
  s = lax.axis_index("s")

  row0 = _zero_table_and_preload(
      zrows_hbm, zrows_v, table_sh, s, psem,
      [(srcc_hbm.at[pl.ds(s * CPT2, CPT2)], src_v),
       (dstc_hbm.at[pl.ds(s * CPT2, CPT2)], dst_v)])
  plsc.subcore_barrier()

  _edge_loop(ys3_hbm.at[c], src_v, dst_v, rows_v, table_sh,
             (g0, g1, g2, g3), (s0, s1, s2, s3), CPT2)
  plsc.subcore_barrier()

  pltpu.async_copy(
      table_sh.at[pl.ds(row0, ROWS_PER_TILE)],
      out_hbm.at[c, pl.ds(row0, ROWS_PER_TILE)],
      osem,
  ).wait()


def _edge_pass2(ys3, srcc, dstc, zrows):
  kfn = pl.kernel(
      _edge2_body,
      out_type=jax.ShapeDtypeStruct((NC, N_PAD, D), jnp.float32),
      mesh=_sc_mesh(),
      scratch_types=[
          pltpu.VMEM((CPT2, CHUNK), jnp.int32),
          pltpu.VMEM((CPT2, CHUNK), jnp.int32),
          pltpu.VMEM((NBUF, CHUNK, D), jnp.float32),
          pltpu.VMEM((128, D), jnp.float32),
          pltpu.VMEM_SHARED((N_PAD, D), jnp.float32),
      ] + [pltpu.SemaphoreType.DMA] * 10,
      compiler_params=pltpu.CompilerParams(use_tc_tiling_on_sc=False),
  )
  return kfn(ys3, srcc, dstc, zrows)


def _tc_scale_matmul_body(degp_ref, x_ref, w_ref, ys_ref, dinv_ref):
  deg = degp_ref[0] + degp_ref[1] + 1.0
  dinv = lax.rsqrt(deg)
  dinv_ref[...] = dinv
  xw = jnp.dot(x_ref[...], w_ref[...], preferred_element_type=jnp.float32)
  ys_ref[...] = xw * dinv


def _tc_scale_matmul(degp, x_pad, w):
  d_out = w.shape[1]
  deg_cols = degp[:, :, 0:1]
  return pl.pallas_call(
      _tc_scale_matmul_body,
      out_shape=(
          jax.ShapeDtypeStruct((N_PAD, d_out), jnp.float32),
          jax.ShapeDtypeStruct((N_PAD, 1), jnp.float32),
      ),
  )(deg_cols, x_pad, w)


def _tc_layer1_finish_body(p_ref, ys_ref, dinv_ref, b_ref, w2_ref, ys3_ref):
  agg = p_ref[0] + p_ref[1] + ys_ref[...]
  h = jnp.maximum(agg * dinv_ref[...] + b_ref[...], 0.0)
  hw = jnp.dot(h, w2_ref[...], preferred_element_type=jnp.float32)
  ys2 = hw * dinv_ref[...]
  ys3_ref[0] = ys2[:, :D]
  ys3_ref[1] = ys2[:, D:]


def _tc_layer1_finish(partials, ys1, dinv, b1, w2):
  return pl.pallas_call(
      _tc_layer1_finish_body,
      out_shape=jax.ShapeDtypeStruct((NC, N_PAD, D), jnp.float32),
  )(partials, ys1, dinv, b1.reshape(1, HID), w2)


def _tc_layer2_finish_body(p_ref, ys3_ref, dinv_ref, b_ref, out_ref):
  agga = p_ref[0] + ys3_ref[0]
  aggb = p_ref[1] + ys3_ref[1]
  out_ref[:, :D] = agga * dinv_ref[...] + b_ref[:, :D]
  out_ref[:, D:] = aggb * dinv_ref[...] + b_ref[:, D:]


def _tc_layer2_finish(p2, ys3, dinv, b2):
  return pl.pallas_call(
      _tc_layer2_finish_body,
      out_shape=jax.ShapeDtypeStruct((N_PAD, OUT_CH), jnp.float32),
  )(p2, ys3, dinv, b2.reshape(1, OUT_CH))


def _gcn_block(x, edge_index, W1, b1, W2, b2):
  src = edge_index[0].astype(jnp.int32)
  dst = edge_index[1].astype(jnp.int32)
  npad_e = E_PAD - N_EDGES
  pad_idx = N_NODES + (jnp.arange(npad_e, dtype=jnp.int32) % (N_PAD - N_NODES))
  both = jnp.concatenate(
      [jnp.stack([src, dst]), jnp.stack([pad_idx, pad_idx])], axis=1)
  srcc = both[0].reshape(NCHUNKS, CHUNK)
  dstc = both[1].reshape(NCHUNKS, CHUNK)

  x_pad = jnp.zeros((N_PAD, IN_CH), x.dtype).at[:N_NODES].set(x)

  ones = jnp.ones((CHUNK, 8), jnp.float32)
  zrows8 = jnp.zeros((128, 8), jnp.float32)
  zrows_d = jnp.zeros((128, D), jnp.float32)

  degp = _deg_pass(dstc, ones, zrows8)

  ys1, dinv = _tc_scale_matmul(degp, x_pad, W1)
  p1 = _edge_pass1(ys1, srcc, dstc, zrows_d)

  ys3 = _tc_layer1_finish(p1, ys1, dinv, b1, W2)
  p2 = _edge_pass2(ys3, srcc, dstc, zrows_d)

  out = _tc_layer2_finish(p2, ys3, dinv, b2)
  return out[:N_NODES]


def kernel(x, edge_index, W1, b1, W2, b2):
  return _gcn_block(x, edge_index, W1, b1, W2, b2)
